# baseline (speedup 1.0000x reference)
import jax
import jax.numpy as jnp
from jax.experimental import pallas as pl
from jax.experimental.pallas import tpu as pltpu

_B = 32
_A = 3
_ATTR = 85
_GW = 76
_S = _GW * _GW
_NP = _B * _A
_STRIDE = 8.0
_ANCH_W = (116.0, 156.0, 373.0)
_ANCH_H = (90.0, 198.0, 326.0)
_DEPTH = 6


def _decode_plane(v, a):
    s_iota = jax.lax.broadcasted_iota(jnp.int32, (1, _S), 1)
    gx = (s_iota % _GW).astype(jnp.float32)
    gy = (s_iota // _GW).astype(jnp.float32)
    aw = jnp.where(a == 0, _ANCH_W[0], jnp.where(a == 1, _ANCH_W[1], _ANCH_W[2]))
    ah = jnp.where(a == 0, _ANCH_H[0], jnp.where(a == 1, _ANCH_H[1], _ANCH_H[2]))
    sig = jax.nn.sigmoid(v)
    row0 = (sig[0:1] + gx) * _STRIDE
    row1 = (sig[1:2] + gy) * _STRIDE
    row2 = jnp.exp(v[2:3]) * aw
    row3 = jnp.exp(v[3:4]) * ah
    t = jnp.concatenate([row0, row1, row2, row3, sig[4:]], axis=0)
    return t.T


def _pipeline(x_hbm, o_hbm, in_buf, out_buf, in_sems, out_sems):
    def in_copy(p, slot):
        return pltpu.make_async_copy(
            x_hbm.at[p], in_buf.at[slot], in_sems.at[slot])

    def out_copy(p, slot):
        return pltpu.make_async_copy(
            out_buf.at[slot], o_hbm.at[p], out_sems.at[slot])

    for p in range(_DEPTH):
        in_copy(p, p).start()

    def body(p, carry):
        slot = jax.lax.rem(p, _DEPTH)
        in_copy(p, slot).wait()
        res = _decode_plane(in_buf[slot], jax.lax.rem(p, _A))

        @pl.when(p >= _DEPTH)
        def _():
            out_copy(p - _DEPTH, slot).wait()

        out_buf[slot] = res
        out_copy(p, slot).start()

        @pl.when(p + _DEPTH < _NP)
        def _():
            in_copy(p + _DEPTH, slot).start()

        return carry

    jax.lax.fori_loop(0, _NP, body, 0)

    for i in range(_DEPTH):
        p = _NP - _DEPTH + i
        out_copy(p, jax.lax.rem(p, _DEPTH)).wait()


def kernel(inputs):
    x3 = inputs.reshape(_NP, _ATTR, _S)
    out3 = pl.pallas_call(
        _pipeline,
        in_specs=[pl.BlockSpec(memory_space=pl.ANY)],
        out_specs=pl.BlockSpec(memory_space=pl.ANY),
        out_shape=jax.ShapeDtypeStruct((_NP, _S, _ATTR), jnp.float32),
        scratch_shapes=[
            pltpu.VMEM((_DEPTH, _ATTR, _S), jnp.float32),
            pltpu.VMEM((_DEPTH, _S, _ATTR), jnp.float32),
            pltpu.SemaphoreType.DMA((_DEPTH,)),
            pltpu.SemaphoreType.DMA((_DEPTH,)),
        ],
    )(x3)
    return out3.reshape(_B, _A * _S, _ATTR)
